# baseline (speedup 1.0000x reference)
import jax
import jax.numpy as jnp
from jax import lax
from jax.experimental import pallas as pl
from jax.experimental.pallas import tpu as pltpu
from jax.experimental.pallas import tpu_sc as plsc

L = 16

B, N, C = 16, 4096, 64
CLUSTER = 1024
NC, NS = 2, 16
B_PER_SC = B // NC
SEGS_SC = B_PER_SC * CLUSTER
TOK_SC = B_PER_SC * N
TOK_TILE = TOK_SC // NS
SUB = 64
NSUB = TOK_TILE // SUB
W = 128
SEGS_TILE = SEGS_SC // NS


def _sc_kernel(x_hbm, idx_hbm, tw_hbm, out_hbm, acc_sh,
               xb0, xb1, xw0, xw1, idxfull, twfull, idxs0, idxs1,
               xsem0, xsem1, ssem0, ssem1):
    c = lax.axis_index("c")
    s = lax.axis_index("s")
    b_local = s // (NS // B_PER_SC)
    boff = b_local * CLUSTER
    tok0 = c * TOK_SC + s * TOK_TILE
    zero16 = jnp.zeros((L,), jnp.float32)

    xb = (xb0, xb1)
    xw = (xw0, xw1)
    idxs = (idxs0, idxs1)
    xsem = (xsem0, xsem1)
    ssem = (ssem0, ssem1)

    def start_load(j, p):
        base = pl.multiple_of(tok0 + j * SUB, SUB)
        pltpu.async_copy(x_hbm.at[pl.ds(base, SUB)], xb[p], xsem[p])

    def wait_load(j, p):
        base = pl.multiple_of(tok0 + j * SUB, SUB)
        pltpu.make_async_copy(x_hbm.at[pl.ds(base, SUB)], xb[p], xsem[p]).wait()

    def wait_scatter(p):
        pltpu.make_async_copy(xw[p], acc_sh.at[idxs[p]], ssem[p]).wait()

    tbase = pl.multiple_of(tok0, SUB)
    meta0 = pltpu.make_async_copy(idx_hbm.at[pl.ds(tbase, TOK_TILE)], idxfull, ssem1)
    meta1 = pltpu.make_async_copy(tw_hbm.at[pl.ds(tbase, TOK_TILE)], twfull, ssem1)
    meta0.start()
    meta1.start()
    start_load(0, 0)
    start_load(1, 1)

    @plsc.parallel_loop(0, SUB * (W // L), unroll=4)
    def zrow(i):
        r = i // (W // L)
        f = i % (W // L)
        xw0[r, pl.ds(f * L, L)] = zero16

    zcopies = [
        pltpu.make_async_copy(
            xw0, acc_sh.at[pl.ds(s * SEGS_TILE + k * SUB, SUB)], ssem0)
        for k in range(SEGS_TILE // SUB)
    ]
    for zc in zcopies:
        zc.start()
    for zc in zcopies:
        zc.wait()
    meta0.wait()
    meta1.wait()

    plsc.subcore_barrier()

    def step(g, _):
        for p in range(2):
            j = 2 * g + p

            @pl.when(g > 0)
            def _():
                wait_scatter(p)
            wait_load(j, p)

            @plsc.parallel_loop(0, SUB // L, unroll=4)
            def body(g16):
                tv = twfull[pl.ds(j * SUB + g16 * L, L)]
                idxs[p][pl.ds(g16 * L, L)] = idxfull[pl.ds(j * SUB + g16 * L, L)] + boff
                for tk in range(L):
                    t = g16 * L + tk
                    wv = jnp.full((L,), tv[tk], jnp.float32)
                    for f in range(C // L):
                        xw[p][t, pl.ds(f * L, L)] = xb[p][t, pl.ds(f * L, L)] * wv
                    xw[p][t, pl.ds(C, L)] = wv

            @pl.when(g < (NSUB // 2) - 1)
            def _():
                start_load(j + 2, p)
            pltpu.async_copy(xw[p], acc_sh.at[idxs[p]], ssem[p], add=True)
        return 0
    lax.fori_loop(0, NSUB // 2, step, 0)
    wait_scatter(0)
    wait_scatter(1)

    plsc.subcore_barrier()

    NH = SEGS_TILE // SUB

    def ep_read(h, p):
        seg0 = s * SEGS_TILE + h * SUB
        return pltpu.make_async_copy(acc_sh.at[pl.ds(seg0, SUB)], xw[p], xsem[p])

    def ep_write(h, p):
        seg0 = pl.multiple_of(c * SEGS_SC + s * SEGS_TILE + h * SUB, SUB)
        return pltpu.make_async_copy(xb[p], out_hbm.at[pl.ds(seg0, SUB)], ssem[p])

    ep_read(0, 0).start()
    ep_read(1, 1).start()
    for h in range(NH):
        p = h & 1
        ep_read(h, p).wait()
        if h >= 2:
            ep_write(h - 2, p).wait()

        @plsc.parallel_loop(0, SUB, unroll=4)
        def nbody(t):
            wrow = xw[p][t, pl.ds(C, L)]
            wv = jnp.full((L,), wrow[0], jnp.float32)
            rv = 1.0 / (wv + 1e-6)
            for f in range(C // L):
                xb[p][t, pl.ds(f * L, L)] = xw[p][t, pl.ds(f * L, L)] * rv

        if h < NH - 2:
            ep_read(h + 2, p).start()
        ep_write(h, p).start()
    ep_write(NH - 2, 0).wait()
    ep_write(NH - 1, 1).wait()


@jax.jit
def _ctm_merge(x2, idx1, tw1):
    mesh = plsc.VectorSubcoreMesh(core_axis_name="c", subcore_axis_name="s")
    run = pl.kernel(
        _sc_kernel,
        out_type=jax.ShapeDtypeStruct((B * CLUSTER, C), jnp.float32),
        mesh=mesh,
        compiler_params=pltpu.CompilerParams(use_tc_tiling_on_sc=True),
        scratch_types=[
            pltpu.VMEM_SHARED((SEGS_SC, W), jnp.float32),
            pltpu.VMEM((SUB, C), jnp.float32),
            pltpu.VMEM((SUB, C), jnp.float32),
            pltpu.VMEM((SUB, W), jnp.float32),
            pltpu.VMEM((SUB, W), jnp.float32),
            pltpu.VMEM((TOK_TILE,), jnp.int32),
            pltpu.VMEM((TOK_TILE,), jnp.float32),
            pltpu.VMEM((SUB,), jnp.int32),
            pltpu.VMEM((SUB,), jnp.int32),
            pltpu.SemaphoreType.DMA,
            pltpu.SemaphoreType.DMA,
            pltpu.SemaphoreType.DMA,
            pltpu.SemaphoreType.DMA,
        ],
    )
    return run(x2, idx1, tw1)


def kernel(x, idx_cluster, token_weight, cluster_num):
    b, n, c = x.shape
    x2 = x.reshape(b * n, c)
    idx1 = idx_cluster.reshape(b * n)
    tw1 = token_weight.reshape(b * n)
    out = _ctm_merge(x2, idx1, tw1)
    return out.reshape(b, CLUSTER, c)


# speedup vs baseline: 1.0083x; 1.0083x over previous
import jax
import jax.numpy as jnp
from jax import lax
from jax.experimental import pallas as pl
from jax.experimental.pallas import tpu as pltpu
from jax.experimental.pallas import tpu_sc as plsc

L = 16

B, N, C = 16, 4096, 64
CLUSTER = 1024
NC, NS = 2, 16
B_PER_SC = B // NC
SEGS_SC = B_PER_SC * CLUSTER
TOK_SC = B_PER_SC * N
TOK_TILE = TOK_SC // NS
SUB = 64
NSUB = TOK_TILE // SUB
W = 128
SEGS_TILE = SEGS_SC // NS


def _sc_kernel(x_hbm, idx_hbm, tw_hbm, out_hbm, acc_sh,
               xb0, xb1, xw0, xw1, idxfull, twfull, idxs0, idxs1,
               xsem0, xsem1, ssem0, ssem1):
    c = lax.axis_index("c")
    s = lax.axis_index("s")
    b_local = s // (NS // B_PER_SC)
    boff = b_local * CLUSTER
    tok0 = c * TOK_SC + s * TOK_TILE
    zero16 = jnp.zeros((L,), jnp.float32)

    xb = (xb0, xb1)
    xw = (xw0, xw1)
    idxs = (idxs0, idxs1)
    xsem = (xsem0, xsem1)
    ssem = (ssem0, ssem1)

    def start_load(j, p):
        base = pl.multiple_of(tok0 + j * SUB, SUB)
        pltpu.async_copy(x_hbm.at[pl.ds(base, SUB)], xb[p], xsem[p])

    def wait_load(j, p):
        base = pl.multiple_of(tok0 + j * SUB, SUB)
        pltpu.make_async_copy(x_hbm.at[pl.ds(base, SUB)], xb[p], xsem[p]).wait()

    def wait_scatter(p):
        pltpu.make_async_copy(xw[p], acc_sh.at[idxs[p]], ssem[p]).wait()

    tbase = pl.multiple_of(tok0, SUB)
    meta0 = pltpu.make_async_copy(idx_hbm.at[pl.ds(tbase, TOK_TILE)], idxfull, ssem1)
    meta1 = pltpu.make_async_copy(tw_hbm.at[pl.ds(tbase, TOK_TILE)], twfull, ssem1)
    meta0.start()
    meta1.start()
    start_load(0, 0)
    start_load(1, 1)

    @plsc.parallel_loop(0, SUB * (W // L), unroll=4)
    def zrow(i):
        r = i // (W // L)
        f = i % (W // L)
        xw0[r, pl.ds(f * L, L)] = zero16

    zcopies = [
        pltpu.make_async_copy(
            xw0, acc_sh.at[pl.ds(s * SEGS_TILE + k * SUB, SUB)], ssem0)
        for k in range(SEGS_TILE // SUB)
    ]
    for zc in zcopies:
        zc.start()
    for zc in zcopies:
        zc.wait()
    meta0.wait()
    meta1.wait()

    plsc.subcore_barrier()

    def step(g, _):
        for p in range(2):
            j = 2 * g + p

            @pl.when(g > 0)
            def _():
                wait_scatter(p)
            wait_load(j, p)

            @plsc.parallel_loop(0, SUB // L, unroll=4)
            def body(g16):
                tv = twfull[pl.ds(j * SUB + g16 * L, L)]
                idxs[p][pl.ds(g16 * L, L)] = idxfull[pl.ds(j * SUB + g16 * L, L)] + boff
                for tk in range(L):
                    t = g16 * L + tk
                    wv = jnp.full((L,), tv[tk], jnp.float32)
                    for f in range(C // L):
                        xw[p][t, pl.ds(f * L, L)] = xb[p][t, pl.ds(f * L, L)] * wv
                    xw[p][t, pl.ds(C, L)] = wv

            @pl.when(g < (NSUB // 2) - 1)
            def _():
                start_load(j + 2, p)
            pltpu.async_copy(xw[p], acc_sh.at[idxs[p]], ssem[p], add=True)
        return 0
    lax.fori_loop(0, NSUB // 2, step, 0)
    wait_scatter(0)
    wait_scatter(1)

    plsc.subcore_barrier()

    NH = SEGS_TILE // SUB

    def ep_read(h, p):
        seg0 = s * SEGS_TILE + h * SUB
        return pltpu.make_async_copy(acc_sh.at[pl.ds(seg0, SUB)], xw[p], xsem[p])

    def ep_write(h, p):
        seg0 = pl.multiple_of(c * SEGS_SC + s * SEGS_TILE + h * SUB, SUB)
        return pltpu.make_async_copy(xb[p], out_hbm.at[pl.ds(seg0, SUB)], ssem[p])

    ep_read(0, 0).start()
    ep_read(1, 1).start()
    for h in range(NH):
        p = h & 1
        ep_read(h, p).wait()
        if h >= 2:
            ep_write(h - 2, p).wait()

        @plsc.parallel_loop(0, SUB, unroll=2)
        def nbody(t):
            wrow = xw[p][t, pl.ds(C, L)]
            wv = jnp.full((L,), wrow[0], jnp.float32)
            rv = 1.0 / (wv + 1e-6)
            for f in range(C // L):
                xb[p][t, pl.ds(f * L, L)] = xw[p][t, pl.ds(f * L, L)] * rv

        if h < NH - 2:
            ep_read(h + 2, p).start()
        ep_write(h, p).start()
    ep_write(NH - 2, 0).wait()
    ep_write(NH - 1, 1).wait()


@jax.jit
def _ctm_merge(x2, idx1, tw1):
    mesh = plsc.VectorSubcoreMesh(core_axis_name="c", subcore_axis_name="s")
    run = pl.kernel(
        _sc_kernel,
        out_type=jax.ShapeDtypeStruct((B * CLUSTER, C), jnp.float32),
        mesh=mesh,
        compiler_params=pltpu.CompilerParams(use_tc_tiling_on_sc=True),
        scratch_types=[
            pltpu.VMEM_SHARED((SEGS_SC, W), jnp.float32),
            pltpu.VMEM((SUB, C), jnp.float32),
            pltpu.VMEM((SUB, C), jnp.float32),
            pltpu.VMEM((SUB, W), jnp.float32),
            pltpu.VMEM((SUB, W), jnp.float32),
            pltpu.VMEM((TOK_TILE,), jnp.int32),
            pltpu.VMEM((TOK_TILE,), jnp.float32),
            pltpu.VMEM((SUB,), jnp.int32),
            pltpu.VMEM((SUB,), jnp.int32),
            pltpu.SemaphoreType.DMA,
            pltpu.SemaphoreType.DMA,
            pltpu.SemaphoreType.DMA,
            pltpu.SemaphoreType.DMA,
        ],
    )
    return run(x2, idx1, tw1)


def kernel(x, idx_cluster, token_weight, cluster_num):
    b, n, c = x.shape
    x2 = x.reshape(b * n, c)
    idx1 = idx_cluster.reshape(b * n)
    tw1 = token_weight.reshape(b * n)
    out = _ctm_merge(x2, idx1, tw1)
    return out.reshape(b, CLUSTER, c)
